# baseline (speedup 1.0000x reference)
import jax
import jax.numpy as jnp
from jax.experimental import pallas as pl
from jax.experimental.pallas import tpu as pltpu


_TOPK = 2
_TPAD = 8
_FB = 2048


def _fused_body(xs_ref, x8_ref, w1_ref, b1_ref, w2_ref, b2_ref, wg_ref, bg_ref,
                o_ref, c_ref, y_ref):
    e = pl.program_id(0)
    f = pl.program_id(1)
    ne = pl.num_programs(0)
    nf = pl.num_programs(1)
    sb = xs_ref.shape[0]

    @pl.when(f == 0)
    def _gating():
        logits = jnp.dot(xs_ref[...], wg_ref[...],
                         preferred_element_type=jnp.float32) + bg_ref[...]
        m = jnp.max(logits, axis=-1, keepdims=True)
        p = jnp.exp(logits - m)
        p = p / jnp.sum(p, axis=-1, keepdims=True)
        a1 = jnp.argmax(p, axis=-1, keepdims=True)
        v1 = jnp.max(p, axis=-1, keepdims=True)
        e_iota = jax.lax.broadcasted_iota(jnp.int32, p.shape, 1)
        p_masked = jnp.where(e_iota == a1, -jnp.inf, p)
        a2 = jnp.argmax(p_masked, axis=-1, keepdims=True)
        v2 = jnp.max(p_masked, axis=-1, keepdims=True)
        t = jax.lax.broadcasted_iota(jnp.int32, (sb, c_ref.shape[1]), 1)
        c = (jnp.where(t == a1 * _TPAD, v1, 0.0)
             + jnp.where(t == a2 * _TPAD + 1, v2, 0.0))
        c_ref[pl.ds(e * sb, sb), :] = c

    h = jnp.dot(x8_ref[...], w1_ref[0], preferred_element_type=jnp.float32)
    h = h + b1_ref[0]
    h = 0.5 * h * (1.0 + jax.lax.erf(h * jnp.float32(0.7071067811865476)))
    contrib = jnp.dot(h, w2_ref[0], preferred_element_type=jnp.float32)

    @pl.when(f == 0)
    def _init():
        y_ref[pl.ds(e * _TPAD, _TPAD), :] = contrib + b2_ref[0]

    @pl.when(f != 0)
    def _acc():
        y_ref[pl.ds(e * _TPAD, _TPAD), :] = (
            y_ref[pl.ds(e * _TPAD, _TPAD), :] + contrib)

    @pl.when((e == ne - 1) & (f == nf - 1))
    def _combine():
        o_ref[...] = jnp.dot(c_ref[...], y_ref[...],
                             preferred_element_type=jnp.float32)


def kernel(x, W1, b1, W2, b2, Wg, bg):
    B, S, D = x.shape
    E, _, F = W1.shape
    x2d = x.reshape(S, D)
    x8 = x2d[:_TPAD]
    b1r = b1.reshape(E, 1, F)
    b2r = b2.reshape(E, 1, D)
    bgr = bg.reshape(1, E)

    nf = F // _FB
    sb = S // E

    out = pl.pallas_call(
        _fused_body,
        grid=(E, nf),
        in_specs=[
            pl.BlockSpec((sb, D), lambda e, f: (e, 0)),
            pl.BlockSpec((_TPAD, D), lambda e, f: (0, 0)),
            pl.BlockSpec((1, D, _FB), lambda e, f: (e, 0, f)),
            pl.BlockSpec((1, 1, _FB), lambda e, f: (e, 0, f)),
            pl.BlockSpec((1, _FB, D), lambda e, f: (e, f, 0)),
            pl.BlockSpec((1, 1, D), lambda e, f: (e, 0, 0)),
            pl.BlockSpec((D, E), lambda e, f: (0, 0)),
            pl.BlockSpec((1, E), lambda e, f: (0, 0)),
        ],
        out_specs=pl.BlockSpec((S, D), lambda e, f: (0, 0)),
        out_shape=jax.ShapeDtypeStruct((S, D), jnp.float32),
        scratch_shapes=[
            pltpu.VMEM((S, E * _TPAD), jnp.float32),
            pltpu.VMEM((E * _TPAD, D), jnp.float32),
        ],
        compiler_params=pltpu.CompilerParams(
            dimension_semantics=("arbitrary", "arbitrary"),
        ),
    )(x2d, x8, W1, b1r, W2, b2r, Wg, bgr)

    return out.reshape(B, S, D)


# speedup vs baseline: 1.0046x; 1.0046x over previous
import jax
import jax.numpy as jnp
from jax.experimental import pallas as pl
from jax.experimental.pallas import tpu as pltpu


_TOPK = 2
_TPAD = 8
_FB = 1024


def _fused_body(xs_ref, x8_ref, w1_ref, b1_ref, w2_ref, b2_ref, wg_ref, bg_ref,
                o_ref, c_ref, y_ref):
    e = pl.program_id(0)
    f = pl.program_id(1)
    ne = pl.num_programs(0)
    nf = pl.num_programs(1)
    sb = xs_ref.shape[0]

    @pl.when(f == 0)
    def _gating():
        logits = jnp.dot(xs_ref[...], wg_ref[...],
                         preferred_element_type=jnp.float32) + bg_ref[...]
        m = jnp.max(logits, axis=-1, keepdims=True)
        p = jnp.exp(logits - m)
        p = p / jnp.sum(p, axis=-1, keepdims=True)
        a1 = jnp.argmax(p, axis=-1, keepdims=True)
        v1 = jnp.max(p, axis=-1, keepdims=True)
        e_iota = jax.lax.broadcasted_iota(jnp.int32, p.shape, 1)
        p_masked = jnp.where(e_iota == a1, -jnp.inf, p)
        a2 = jnp.argmax(p_masked, axis=-1, keepdims=True)
        v2 = jnp.max(p_masked, axis=-1, keepdims=True)
        t = jax.lax.broadcasted_iota(jnp.int32, (sb, c_ref.shape[1]), 1)
        c = (jnp.where(t == a1 * _TPAD, v1, 0.0)
             + jnp.where(t == a2 * _TPAD + 1, v2, 0.0))
        c_ref[pl.ds(e * sb, sb), :] = c

    h = jnp.dot(x8_ref[...], w1_ref[0], preferred_element_type=jnp.float32)
    h = h + b1_ref[0]
    h = 0.5 * h * (1.0 + jax.lax.erf(h * jnp.float32(0.7071067811865476)))
    contrib = jnp.dot(h, w2_ref[0], preferred_element_type=jnp.float32)

    @pl.when(f == 0)
    def _init():
        y_ref[pl.ds(e * _TPAD, _TPAD), :] = contrib + b2_ref[0]

    @pl.when(f != 0)
    def _acc():
        y_ref[pl.ds(e * _TPAD, _TPAD), :] = (
            y_ref[pl.ds(e * _TPAD, _TPAD), :] + contrib)

    @pl.when((e == ne - 1) & (f == nf - 1))
    def _combine():
        o_ref[...] = jnp.dot(c_ref[...], y_ref[...],
                             preferred_element_type=jnp.float32)


def kernel(x, W1, b1, W2, b2, Wg, bg):
    B, S, D = x.shape
    E, _, F = W1.shape
    x2d = x.reshape(S, D)
    x8 = x2d[:_TPAD]
    b1r = b1.reshape(E, 1, F)
    b2r = b2.reshape(E, 1, D)
    bgr = bg.reshape(1, E)

    nf = F // _FB
    sb = S // E

    out = pl.pallas_call(
        _fused_body,
        grid=(E, nf),
        in_specs=[
            pl.BlockSpec((sb, D), lambda e, f: (e, 0)),
            pl.BlockSpec((_TPAD, D), lambda e, f: (0, 0)),
            pl.BlockSpec((1, D, _FB), lambda e, f: (e, 0, f)),
            pl.BlockSpec((1, 1, _FB), lambda e, f: (e, 0, f)),
            pl.BlockSpec((1, _FB, D), lambda e, f: (e, f, 0)),
            pl.BlockSpec((1, 1, D), lambda e, f: (e, 0, 0)),
            pl.BlockSpec((D, E), lambda e, f: (0, 0)),
            pl.BlockSpec((1, E), lambda e, f: (0, 0)),
        ],
        out_specs=pl.BlockSpec((S, D), lambda e, f: (0, 0)),
        out_shape=jax.ShapeDtypeStruct((S, D), jnp.float32),
        scratch_shapes=[
            pltpu.VMEM((S, E * _TPAD), jnp.float32),
            pltpu.VMEM((E * _TPAD, D), jnp.float32),
        ],
        compiler_params=pltpu.CompilerParams(
            dimension_semantics=("arbitrary", "arbitrary"),
        ),
    )(x2d, x8, W1, b1r, W2, b2r, Wg, bgr)

    return out.reshape(B, S, D)


# speedup vs baseline: 1.0050x; 1.0004x over previous
import jax
import jax.numpy as jnp
from jax.experimental import pallas as pl
from jax.experimental.pallas import tpu as pltpu


_TOPK = 2
_TPAD = 8
_FB = 1024


def _fused_body(xs_ref, x8_ref, w1_ref, b1_ref, w2_ref, b2_ref, wg_ref, bg_ref,
                o_ref, c_ref, y_ref):
    e = pl.program_id(0)
    f = pl.program_id(1)
    ne = pl.num_programs(0) - 1
    sb = xs_ref.shape[0]

    @pl.when((f == 0) & (e < ne))
    def _gating():
        logits = jnp.dot(xs_ref[...], wg_ref[...],
                         preferred_element_type=jnp.float32) + bg_ref[...]
        m = jnp.max(logits, axis=-1, keepdims=True)
        p = jnp.exp(logits - m)
        p = p / jnp.sum(p, axis=-1, keepdims=True)
        a1 = jnp.argmax(p, axis=-1, keepdims=True)
        v1 = jnp.max(p, axis=-1, keepdims=True)
        e_iota = jax.lax.broadcasted_iota(jnp.int32, p.shape, 1)
        p_masked = jnp.where(e_iota == a1, -jnp.inf, p)
        a2 = jnp.argmax(p_masked, axis=-1, keepdims=True)
        v2 = jnp.max(p_masked, axis=-1, keepdims=True)
        t = jax.lax.broadcasted_iota(jnp.int32, (sb, c_ref.shape[1]), 1)
        c = (jnp.where(t == a1 * _TPAD, v1, 0.0)
             + jnp.where(t == a2 * _TPAD + 1, v2, 0.0))
        c_ref[pl.ds(e * sb, sb), :] = c

    @pl.when(e < ne)
    def _ffn():
        h = jnp.dot(x8_ref[...], w1_ref[0], preferred_element_type=jnp.float32)
        h = h + b1_ref[0]
        h = 0.5 * h * (1.0 + jax.lax.erf(h * jnp.float32(0.7071067811865476)))
        contrib = jnp.dot(h, w2_ref[0], preferred_element_type=jnp.float32)

        @pl.when(f == 0)
        def _init():
            y_ref[pl.ds(e * _TPAD, _TPAD), :] = contrib + b2_ref[0]

        @pl.when(f != 0)
        def _acc():
            y_ref[pl.ds(e * _TPAD, _TPAD), :] = (
                y_ref[pl.ds(e * _TPAD, _TPAD), :] + contrib)

    @pl.when(e == ne)
    def _combine():
        cs = o_ref.shape[0]
        o_ref[...] = jnp.dot(c_ref[pl.ds(f * cs, cs), :], y_ref[...],
                             preferred_element_type=jnp.float32)


def kernel(x, W1, b1, W2, b2, Wg, bg):
    B, S, D = x.shape
    E, _, F = W1.shape
    x2d = x.reshape(S, D)
    x8 = x2d[:_TPAD]
    b1r = b1.reshape(E, 1, F)
    b2r = b2.reshape(E, 1, D)
    bgr = bg.reshape(1, E)

    nf = F // _FB
    sb = S // E
    cs = S // nf

    def _we(e, f):
        return jnp.minimum(e, E - 1)

    def _wf(e, f):
        return jnp.where(e == E, nf - 1, f)

    out = pl.pallas_call(
        _fused_body,
        grid=(E + 1, nf),
        in_specs=[
            pl.BlockSpec((sb, D), lambda e, f: (jnp.minimum(e, E - 1), 0)),
            pl.BlockSpec((_TPAD, D), lambda e, f: (0, 0)),
            pl.BlockSpec((1, D, _FB), lambda e, f: (_we(e, f), 0, _wf(e, f))),
            pl.BlockSpec((1, 1, _FB), lambda e, f: (_we(e, f), 0, _wf(e, f))),
            pl.BlockSpec((1, _FB, D), lambda e, f: (_we(e, f), _wf(e, f), 0)),
            pl.BlockSpec((1, 1, D), lambda e, f: (_we(e, f), 0, 0)),
            pl.BlockSpec((D, E), lambda e, f: (0, 0)),
            pl.BlockSpec((1, E), lambda e, f: (0, 0)),
        ],
        out_specs=pl.BlockSpec(
            (cs, D), lambda e, f: (jnp.where(e == E, f, 0), 0)),
        out_shape=jax.ShapeDtypeStruct((S, D), jnp.float32),
        scratch_shapes=[
            pltpu.VMEM((S, E * _TPAD), jnp.float32),
            pltpu.VMEM((E * _TPAD, D), jnp.float32),
        ],
        compiler_params=pltpu.CompilerParams(
            dimension_semantics=("arbitrary", "arbitrary"),
        ),
    )(x2d, x8, W1, b1r, W2, b2r, Wg, bgr)

    return out.reshape(B, S, D)
